# Initial kernel scaffold; baseline (speedup 1.0000x reference)
import jax
import jax.numpy as jnp
from jax.experimental import pallas as pl


def kernel(patch_x, nodes_x, W1, b1, W2, b2, subgraphs_batch, subgraphs_nodes_mapper):
    raise NotImplementedError("write your pallas kernel here")


# speedup vs baseline: 61.2134x; 61.2134x over previous
import functools

import jax
import jax.numpy as jnp
from jax import lax
from jax.experimental import pallas as pl
from jax.experimental.pallas import tpu as pltpu
from jax.experimental.pallas import tpu_sc as plsc

_B, _T = 2, 12
_N_PATCH, _N_NODES, _E = 2000, 10000, 160000
_F_P, _F_N, _HORIZON = 16, 16, 12
_IN_DIM = _F_P * _T + _F_N * _T
_G = 128
_NG = _IN_DIM // _G

_NC, _NS = 2, 16
_CHUNK = 128
_STEPS = -(-_E // (_NS * _CHUNK))
_EPAD = _NS * _STEPS * _CHUNK
_NROWS = 10240
_RPT = _NROWS // _NS
_SPLIT = (_STEPS + 1) // 2

_TN = 400
_NT = _N_NODES // _TN


def _sc_segment_sum(p2g, sb_g, sm3):
    mesh = plsc.VectorSubcoreMesh(
        core_axis_name="c", subcore_axis_name="s",
        num_cores=_NC, num_subcores=_NS,
    )

    @functools.partial(
        pl.kernel,
        out_type=[
            jax.ShapeDtypeStruct((_NROWS, _IN_DIM), jnp.float32),
            jax.ShapeDtypeStruct((_NROWS, _G), jnp.float32),
            jax.ShapeDtypeStruct((_NROWS,), jnp.float32),
        ],
        mesh=mesh,
        scratch_types=[
            pltpu.VMEM((_STEPS, _CHUNK), jnp.int32),
            pltpu.VMEM((_STEPS, _CHUNK), jnp.int32),
            pltpu.VMEM((_CHUNK, _G), jnp.float32),
            pltpu.VMEM((_CHUNK,), jnp.float32),
            pltpu.VMEM_SHARED((_NROWS, _G), jnp.float32),
            pltpu.VMEM_SHARED((_NROWS,), jnp.float32),
            pltpu.SemaphoreType.DMA,
        ],
        compiler_params=pltpu.CompilerParams(use_tc_tiling_on_sc=False),
    )
    def k(p2g_h, sbg_h, sm3_h, sums_h, part2_h, counts_h,
          idxb, idxm, g, ones_v, acc, cnt, sem):
        c = lax.axis_index("c")
        s = lax.axis_index("s")
        r0 = s * _RPT

        def zrow(r, carry):
            for kk in range(_G // 16):
                g[r, pl.ds(kk * 16, 16)] = jnp.zeros((16,), jnp.float32)
            return carry
        lax.fori_loop(0, _CHUNK, zrow, 0)
        for kk in range(_CHUNK // 16):
            ones_v[pl.ds(kk * 16, 16)] = jnp.ones((16,), jnp.float32)

        for i in range(_RPT // _CHUNK):
            pltpu.sync_copy(g, acc.at[pl.ds(r0 + i * _CHUNK, _CHUNK)])
            pltpu.sync_copy(g.at[0], cnt.at[pl.ds(r0 + i * _CHUNK, _CHUNK)])

        pltpu.sync_copy(sm3_h.at[s], idxm)
        plsc.subcore_barrier()

        def accumulate(j, carry):
            pltpu.async_copy(p2g_h.at[idxb.at[j]], g, sem).wait()
            pltpu.sync_copy(g, acc.at[idxm.at[j]], add=True)
            return carry

        pltpu.sync_copy(sbg_h.at[c, s], idxb)
        lax.fori_loop(0, _STEPS, accumulate, 0)

        def count_step(j, carry):
            pltpu.sync_copy(ones_v, cnt.at[idxm.at[j]], add=True)
            return carry
        lax.fori_loop(0, _STEPS, count_step, 0)

        plsc.subcore_barrier()
        pltpu.sync_copy(
            acc.at[pl.ds(r0, _RPT)],
            sums_h.at[pl.ds(r0, _RPT), pl.ds(c * _G, _G)],
        )

        @pl.when(c == 0)
        def _():
            pltpu.sync_copy(cnt.at[pl.ds(r0, _RPT)], counts_h.at[pl.ds(r0, _RPT)])

        plsc.subcore_barrier()

        lax.fori_loop(0, _CHUNK, zrow, 0)
        for i in range(_RPT // _CHUNK):
            pltpu.sync_copy(g, acc.at[pl.ds(r0 + i * _CHUNK, _CHUNK)])
        pltpu.sync_copy(sbg_h.at[2, s], idxb)
        plsc.subcore_barrier()

        lo = jnp.where(c == 0, 0, _SPLIT)
        hi = jnp.where(c == 0, _SPLIT, _STEPS)
        lax.fori_loop(lo, hi, accumulate, 0)

        plsc.subcore_barrier()

        @pl.when(c == 0)
        def _():
            pltpu.sync_copy(
                acc.at[pl.ds(r0, _RPT)],
                sums_h.at[pl.ds(r0, _RPT), pl.ds(2 * _G, _G)],
            )

        @pl.when(c == 1)
        def _():
            pltpu.sync_copy(acc.at[pl.ds(r0, _RPT)], part2_h.at[pl.ds(r0, _RPT)])

    return k(p2g, sb_g, sm3)


def _mlp_block(sums_ref, part2_ref, cnt_ref, nodes_ref,
               w1_ref, b1_ref, w2_ref, b2_ref, out_ref):
    b = pl.program_id(0)
    full = jnp.concatenate(
        [sums_ref[:, : 2 * _G], sums_ref[:, 2 * _G:] + part2_ref[...]], axis=1)
    inv = 1.0 / jnp.maximum(cnt_ref[...], 1.0)
    p = jnp.where(b == 0, full[:, :_IN_DIM // 2], full[:, _IN_DIM // 2:]) * inv
    x = jnp.concatenate([nodes_ref[0], p], axis=1)
    h = jnp.maximum(x @ w1_ref[...] + b1_ref[...], 0.0)
    out_ref[0] = h @ w2_ref[...] + b2_ref[...]


def _tc_mlp(sums, part2, counts, n2, W1, b1, W2, b2):
    return pl.pallas_call(
        _mlp_block,
        grid=(_B, _NT),
        in_specs=[
            pl.BlockSpec((_TN, _IN_DIM), lambda b, i: (i, 0)),
            pl.BlockSpec((_TN, _G), lambda b, i: (i, 0)),
            pl.BlockSpec((_TN, 1), lambda b, i: (i, 0)),
            pl.BlockSpec((1, _TN, _IN_DIM // 2), lambda b, i: (b, i, 0)),
            pl.BlockSpec((_IN_DIM, _IN_DIM), lambda b, i: (0, 0)),
            pl.BlockSpec((1, _IN_DIM), lambda b, i: (0, 0)),
            pl.BlockSpec((_IN_DIM, _HORIZON), lambda b, i: (0, 0)),
            pl.BlockSpec((1, _HORIZON), lambda b, i: (0, 0)),
        ],
        out_specs=pl.BlockSpec((1, _TN, _HORIZON), lambda b, i: (b, i, 0)),
        out_shape=jax.ShapeDtypeStruct((_B, _N_NODES, _HORIZON), jnp.float32),
    )(sums, part2, counts, n2, W1, b1, W2, b2)


def kernel(patch_x, nodes_x, W1, b1, W2, b2, subgraphs_batch, subgraphs_nodes_mapper):
    p2 = jnp.transpose(patch_x, (2, 0, 1, 3)).reshape(_N_PATCH, _IN_DIM)
    p2g = jnp.concatenate([p2[:, i * _G:(i + 1) * _G] for i in range(_NG)],
                          axis=0)
    n2 = jnp.transpose(nodes_x, (0, 2, 1, 3)).reshape(_B, _N_NODES, _T * _F_N)

    sb = jnp.pad(subgraphs_batch, (0, _EPAD - _E))
    sm = jnp.pad(subgraphs_nodes_mapper, (0, _EPAD - _E),
                 constant_values=_N_NODES)
    sb_g = jnp.stack([sb + i * _N_PATCH for i in range(_NG)]).reshape(
        _NG, _NS, _STEPS, _CHUNK)
    sm3 = sm.reshape(_NS, _STEPS, _CHUNK)

    sums, part2, counts = _sc_segment_sum(p2g, sb_g, sm3)

    out = _tc_mlp(
        sums,
        part2,
        counts.reshape(_NROWS, 1),
        n2,
        W1,
        b1.reshape(1, _IN_DIM),
        W2,
        b2.reshape(1, _HORIZON),
    )
    return out
